# baseline (speedup 1.0000x reference)
import functools

import jax
import jax.numpy as jnp
from jax import lax
from jax.experimental import pallas as pl
from jax.experimental.pallas import tpu as pltpu
from jax.experimental.pallas import tpu_sc as plsc

VOCAB_SIZE = 1_000_000
D = 64
B = 16384

_NC = 2
_NS = 16
_L = 16
_NW = _NC * _NS
_BPW = B // _NW
_CHUNK = 256
_NCHUNK = _BPW // _CHUNK
_CGROUPS = _CHUNK // _L


@functools.partial(
    pl.kernel,
    out_type=jax.ShapeDtypeStruct((B,), jnp.float32),
    mesh=plsc.VectorSubcoreMesh(core_axis_name="c", subcore_axis_name="s"),
    scratch_types=[
        pltpu.VMEM((_BPW,), jnp.int32),
        pltpu.VMEM((_BPW,), jnp.int32),
        pltpu.VMEM((_BPW,), jnp.int32),
        pltpu.VMEM((_BPW,), jnp.int32),
        pltpu.VMEM((_CHUNK, 2 * D), jnp.float32),
        pltpu.VMEM((_CHUNK, 2 * D), jnp.float32),
        pltpu.VMEM((_BPW,), jnp.float32),
        pltpu.SemaphoreType.DMA,
        pltpu.SemaphoreType.DMA,
    ],
    compiler_params=pltpu.CompilerParams(needs_layout_passes=False),
)
def _w2v_kernel(tgt_hbm, ctx_hbm, win_hbm, wout_hbm, out_hbm,
                idx_t, idx_c, pid_t, pid_c, rows_t, rows_c, outv,
                sem_t, sem_c):
    wid = lax.axis_index("s") * _NC + lax.axis_index("c")
    base = wid * _BPW

    pltpu.sync_copy(tgt_hbm.at[pl.ds(base, _BPW)], idx_t)
    pltpu.sync_copy(ctx_hbm.at[pl.ds(base, _BPW)], idx_c)

    lane = lax.iota(jnp.int32, _L)

    def pid_body(i, carry):
        sl = pl.ds(i * _L, _L)
        pid_t[sl] = lax.shift_right_logical(idx_t[sl], 1)
        pid_c[sl] = lax.shift_right_logical(idx_c[sl], 1)
        return carry

    lax.fori_loop(0, _BPW // _L, pid_body, 0)

    for chunk in range(_NCHUNK):
        cbase = chunk * _CHUNK
        cp_t = pltpu.async_copy(
            win_hbm.at[pid_t.at[pl.ds(cbase, _CHUNK)]], rows_t, sem_t)
        cp_c = pltpu.async_copy(
            wout_hbm.at[pid_c.at[pl.ds(cbase, _CHUNK)]], rows_c, sem_c)
        cp_t.wait()
        cp_c.wait()

        def group_body(g, carry):
            rvec = g * _L + lane
            toff = (idx_t[pl.ds(cbase + g * _L, _L)] & 1) * D
            coff = (idx_c[pl.ds(cbase + g * _L, _L)] & 1) * D
            acc = jnp.zeros((_L,), jnp.float32)
            for j in range(D):
                t = plsc.load_gather(rows_t, [rvec, toff + j])
                c = plsc.load_gather(rows_c, [rvec, coff + j])
                acc = acc + t * c
            outv[pl.ds(cbase + g * _L, _L)] = acc
            return carry

        lax.fori_loop(0, _CGROUPS, group_body, 0)

    pltpu.sync_copy(outv, out_hbm.at[pl.ds(base, _BPW)])


def kernel(target_word, context_word, W_in, W_out):
    return _w2v_kernel(target_word.astype(jnp.int32),
                       context_word.astype(jnp.int32),
                       W_in.reshape(VOCAB_SIZE // 2, 2 * D),
                       W_out.reshape(VOCAB_SIZE // 2, 2 * D))


# speedup vs baseline: 1.9657x; 1.9657x over previous
import functools

import jax
import jax.numpy as jnp
from jax import lax
from jax.experimental import pallas as pl
from jax.experimental.pallas import tpu as pltpu
from jax.experimental.pallas import tpu_sc as plsc

VOCAB_SIZE = 1_000_000
D = 64
B = 16384

_NC = 2
_NS = 16
_L = 16
_NW = _NC * _NS
_VTILE = 128
_NTILES = 7812
_VCUT = _NTILES * _VTILE
_BASE_T = _NTILES // _NW
_EXTRA = _NTILES - _BASE_T * _NW
_WIN = 512
_PAD_IDX = 1 << 20
_OUTROWS = B + 128


def _worker_starts():
    starts = []
    t = 0
    for w in range(_NW + 1):
        starts.append(t * _VTILE)
        t += _BASE_T + (1 if w < _EXTRA else 0)
    starts[_NW] = _VCUT
    return starts


@functools.partial(
    pl.kernel,
    out_type=(
        jax.ShapeDtypeStruct((_OUTROWS, 2 * D), jnp.float32),
        jax.ShapeDtypeStruct((_OUTROWS, 2 * D), jnp.float32),
    ),
    mesh=plsc.VectorSubcoreMesh(core_axis_name="c", subcore_axis_name="s"),
    scratch_types=[
        pltpu.VMEM_SHARED((_NS, _WIN), jnp.int32),
        pltpu.SMEM((_WIN,), jnp.int32),
        pltpu.SMEM((_WIN,), jnp.int32),
        pltpu.VMEM((2, 8, 8, _VTILE), jnp.float32),
        pltpu.VMEM((8, 8, D), jnp.float32),
        pltpu.VMEM((_L, 2 * D), jnp.float32),
        pltpu.SemaphoreType.DMA,
        pltpu.SemaphoreType.DMA,
    ],
    compiler_params=pltpu.CompilerParams(needs_layout_passes=False),
)
def _stream_kernel(sidx_t_hbm, spos_t_hbm, p0_t_hbm,
                   sidx_c_hbm, spos_c_hbm, p0_c_hbm,
                   xin_hbm, xout_hbm, trows_hbm, crows_hbm,
                   sp_hop, win_idx, win_pos, slab, tail_slab, rowblk,
                   sem_slab, sem_out):
    sid = lax.axis_index("s")
    wid = sid * _NC + lax.axis_index("c")
    lane = lax.iota(jnp.int32, _L)
    kvecs = [((jnp.arange(16) + 16 * k) // 8, (jnp.arange(16) + 16 * k) % 8)
             for k in range(4)]
    start_tile = wid * _BASE_T + jnp.minimum(wid, _EXTRA)
    ntiles = _BASE_T + (wid < _EXTRA).astype(jnp.int32)
    trash = jnp.full((_L,), B, jnp.int32)

    def run_pass(table_hbm, sidx_hbm, spos_hbm, p0_hbm, out_hbm):
        pltpu.sync_copy(p0_hbm, sp_hop.at[sid])
        pltpu.sync_copy(sp_hop.at[sid], win_idx)
        p_begin = win_idx[wid]
        p_end = win_idx[wid + 1]

        def refill(pbase):
            pbase = pl.multiple_of(pbase, 128)
            pltpu.sync_copy(sidx_hbm.at[pl.ds(pbase, _WIN)], sp_hop.at[sid])
            pltpu.sync_copy(sp_hop.at[sid], win_idx)
            pltpu.sync_copy(spos_hbm.at[pl.ds(pbase, _WIN)], sp_hop.at[sid])
            pltpu.sync_copy(sp_hop.at[sid], win_pos)

        pbase0 = p_begin & -128
        refill(pbase0)

        def slab_src(tno):
            v0 = (start_tile + tno) * _VTILE
            return table_hbm.at[:, :, pl.ds(v0, _VTILE)]

        def flush(pvec):
            pltpu.async_copy(rowblk, out_hbm.at[pvec], sem_out)
            pltpu.make_async_copy(rowblk, out_hbm.at[pvec], sem_out).wait()

        pltpu.async_copy(slab_src(0), slab.at[0], sem_slab)

        def extract_range(slab_ref, v0, v_end, p_limit, carry):
            def cond(c):
                pl_o, pb, _, _ = c
                return (pb + pl_o < p_limit) & (win_idx[pl_o] < v_end)

            def body(c):
                pl_o, pb, m, pv = c
                off = jnp.full((_L,), win_idx[pl_o] - v0, jnp.int32)
                for k, (ivec, svec) in enumerate(kvecs):
                    v = plsc.load_gather(slab_ref, [ivec, svec, off])
                    rowblk[m, pl.ds(k * _L, _L)] = v
                pv = jnp.where(lane == m, win_pos[pl_o], pv)
                m = m + 1
                full = m == _L

                @pl.when(full)
                def _():
                    flush(pv)

                m = jnp.where(full, 0, m)
                pl_o = pl_o + 1
                need = pl_o >= _WIN

                @pl.when(need)
                def _():
                    refill(pb + _WIN)

                pl_o = jnp.where(need, pl_o - _WIN, pl_o)
                pb = jnp.where(need, pb + _WIN, pb)
                return pl_o, pb, m, pv

            return lax.while_loop(cond, body, carry)

        def loop_body(g, carry):
            for b in range(2):
                tno = g * 2 + b

                @pl.when(tno < ntiles)
                def _():
                    pltpu.make_async_copy(
                        slab_src(tno), slab.at[b], sem_slab).wait()

                @pl.when(tno + 1 < ntiles)
                def _():
                    pltpu.async_copy(
                        slab_src(tno + 1), slab.at[1 - b], sem_slab)

                v_end = (start_tile + tno + 1) * _VTILE
                carry = lax.cond(
                    tno < ntiles,
                    lambda c, ve=v_end, bb=b: extract_range(
                        slab.at[bb], ve - _VTILE, ve, p_end, c),
                    lambda c: c,
                    carry)
            return carry

        init = (p_begin - pbase0, pbase0, jnp.int32(0), trash)
        carry = lax.fori_loop(0, (_BASE_T + 2) // 2, loop_body, init)

        def tail_fn(c):
            pltpu.sync_copy(table_hbm.at[:, :, pl.ds(_VCUT, D)], tail_slab)
            return extract_range(tail_slab, _VCUT, VOCAB_SIZE, B, c)

        carry = lax.cond(wid == _NW - 1, tail_fn, lambda c: c, carry)

        _, _, m16, pvec = carry
        pvec = jnp.where(lane < m16, pvec, B)
        flush(pvec)

    run_pass(xin_hbm, sidx_t_hbm, spos_t_hbm, p0_t_hbm, trows_hbm)
    run_pass(xout_hbm, sidx_c_hbm, spos_c_hbm, p0_c_hbm, crows_hbm)


def _dot_body(t_ref, c_ref, o_ref):
    t = t_ref[:, :D]
    c = c_ref[:, :D]
    ones = jnp.ones((D, 1), jnp.float32)
    o_ref[...] = jax.lax.dot(t * c, ones)


_GRID2 = 4
_BLK2 = _OUTROWS // _GRID2
_dot_kernel = pl.pallas_call(
    _dot_body,
    out_shape=jax.ShapeDtypeStruct((_OUTROWS, 1), jnp.float32),
    grid=(_GRID2,),
    in_specs=[
        pl.BlockSpec((_BLK2, 2 * D), lambda i: (i, 0)),
        pl.BlockSpec((_BLK2, 2 * D), lambda i: (i, 0)),
    ],
    out_specs=pl.BlockSpec((_BLK2, 1), lambda i: (i, 0)),
)


def kernel(target_word, context_word, W_in, W_out):
    idx_t = target_word.astype(jnp.int32)
    idx_c = context_word.astype(jnp.int32)
    xin = W_in.T.reshape(8, 8, VOCAB_SIZE)
    xout = W_out.T.reshape(8, 8, VOCAB_SIZE)

    pos = jnp.arange(B, dtype=jnp.int32)
    sidx_t, spos_t = lax.sort((idx_t, pos), num_keys=1)
    sidx_c, spos_c = lax.sort((idx_c, pos), num_keys=1)
    bounds = jnp.asarray(_worker_starts(), dtype=jnp.int32)
    p0_t = jnp.pad(jnp.searchsorted(sidx_t, bounds).astype(jnp.int32),
                   (0, _WIN - _NW - 1))
    p0_c = jnp.pad(jnp.searchsorted(sidx_c, bounds).astype(jnp.int32),
                   (0, _WIN - _NW - 1))
    pad_i = jnp.full((_WIN,), _PAD_IDX, jnp.int32)
    pad_p = jnp.zeros((_WIN,), jnp.int32)
    sidx_t = jnp.concatenate([sidx_t, pad_i])
    spos_t = jnp.concatenate([spos_t, pad_p])
    sidx_c = jnp.concatenate([sidx_c, pad_i])
    spos_c = jnp.concatenate([spos_c, pad_p])

    trows, crows = _stream_kernel(sidx_t, spos_t, p0_t,
                                  sidx_c, spos_c, p0_c, xin, xout)

    return _dot_kernel(trows, crows)[:B, 0]


# speedup vs baseline: 3.4063x; 1.7329x over previous
import functools

import jax
import jax.numpy as jnp
from jax import lax
from jax.experimental import pallas as pl
from jax.experimental.pallas import tpu as pltpu
from jax.experimental.pallas import tpu_sc as plsc

VOCAB_SIZE = 1_000_000
D = 64
B = 16384

_NC = 2
_NS = 16
_L = 16
_NW = _NC * _NS
_VTILE = 128
_NTILES = 7812
_VCUT = _NTILES * _VTILE
_BASE_T = _NTILES // _NW
_EXTRA = _NTILES - _BASE_T * _NW
_WIN = 512
_PAD_IDX = 1 << 20
_OUTROWS = B + 128


def _worker_starts():
    starts = []
    t = 0
    for w in range(_NW + 1):
        starts.append(t * _VTILE)
        t += _BASE_T + (1 if w < _EXTRA else 0)
    starts[_NW] = _VCUT
    return starts


@functools.partial(
    pl.kernel,
    out_type=(
        jax.ShapeDtypeStruct((_OUTROWS, 2 * D), jnp.float32),
        jax.ShapeDtypeStruct((_OUTROWS, 2 * D), jnp.float32),
    ),
    mesh=plsc.VectorSubcoreMesh(core_axis_name="c", subcore_axis_name="s"),
    scratch_types=[
        pltpu.VMEM_SHARED((_NS, _WIN), jnp.int32),
        pltpu.SMEM((_WIN,), jnp.int32),
        pltpu.SMEM((_WIN,), jnp.int32),
        pltpu.VMEM((4, 8, 8, _VTILE), jnp.float32),
        pltpu.VMEM((8, 8, D), jnp.float32),
        pltpu.VMEM((_L, 2 * D), jnp.float32),
        pltpu.SemaphoreType.DMA,
        pltpu.SemaphoreType.DMA,
    ],
    compiler_params=pltpu.CompilerParams(needs_layout_passes=False),
)
def _stream_kernel(sidx_t_hbm, spos_t_hbm, p0_t_hbm,
                   sidx_c_hbm, spos_c_hbm, p0_c_hbm,
                   xin_hbm, xout_hbm, trows_hbm, crows_hbm,
                   sp_hop, win_idx, win_pos, slab, tail_slab, rowblk,
                   sem_slab, sem_out):
    sid = lax.axis_index("s")
    wid = sid * _NC + lax.axis_index("c")
    lane = lax.iota(jnp.int32, _L)
    kvecs = [((jnp.arange(16) + 16 * k) // 8, (jnp.arange(16) + 16 * k) % 8)
             for k in range(4)]
    start_tile = wid * _BASE_T + jnp.minimum(wid, _EXTRA)
    ntiles = _BASE_T + (wid < _EXTRA).astype(jnp.int32)
    trash = jnp.full((_L,), B, jnp.int32)

    def run_pass(table_hbm, sidx_hbm, spos_hbm, p0_hbm, out_hbm):
        pltpu.sync_copy(p0_hbm, sp_hop.at[sid])
        pltpu.sync_copy(sp_hop.at[sid], win_idx)
        p_begin = win_idx[wid]
        p_end = win_idx[wid + 1]

        def refill(pbase):
            pbase = pl.multiple_of(pbase, 128)
            pltpu.sync_copy(sidx_hbm.at[pl.ds(pbase, _WIN)], sp_hop.at[sid])
            pltpu.sync_copy(sp_hop.at[sid], win_idx)
            pltpu.sync_copy(spos_hbm.at[pl.ds(pbase, _WIN)], sp_hop.at[sid])
            pltpu.sync_copy(sp_hop.at[sid], win_pos)

        pbase0 = p_begin & -128
        refill(pbase0)

        def slab_src(tno):
            v0 = (start_tile + tno) * _VTILE
            return table_hbm.at[:, :, pl.ds(v0, _VTILE)]

        def flush(pvec):
            pltpu.async_copy(rowblk, out_hbm.at[pvec], sem_out)
            pltpu.make_async_copy(rowblk, out_hbm.at[pvec], sem_out).wait()

        for b0 in range(4):
            pltpu.async_copy(slab_src(b0), slab.at[b0], sem_slab)

        def extract_range(slab_ref, v0, v_end, p_limit, carry):
            def cond(c):
                pl_o, pb, _, _ = c
                return (pb + pl_o < p_limit) & (win_idx[pl_o] < v_end)

            def body(c):
                pl_o, pb, m, pv = c
                off = jnp.full((_L,), win_idx[pl_o] - v0, jnp.int32)
                for k, (ivec, svec) in enumerate(kvecs):
                    v = plsc.load_gather(slab_ref, [ivec, svec, off])
                    rowblk[m, pl.ds(k * _L, _L)] = v
                pv = jnp.where(lane == m, win_pos[pl_o], pv)
                m = m + 1
                full = m == _L

                @pl.when(full)
                def _():
                    flush(pv)

                m = jnp.where(full, 0, m)
                pl_o = pl_o + 1
                need = pl_o >= _WIN

                @pl.when(need)
                def _():
                    refill(pb + _WIN)

                pl_o = jnp.where(need, pl_o - _WIN, pl_o)
                pb = jnp.where(need, pb + _WIN, pb)
                return pl_o, pb, m, pv

            return lax.while_loop(cond, body, carry)

        def loop_body(g, carry):
            for b in range(4):
                tno = g * 4 + b

                @pl.when(tno < ntiles)
                def _():
                    pltpu.make_async_copy(
                        slab_src(tno), slab.at[b], sem_slab).wait()

                v_end = (start_tile + tno + 1) * _VTILE
                carry = lax.cond(
                    tno < ntiles,
                    lambda c, ve=v_end, bb=b: extract_range(
                        slab.at[bb], ve - _VTILE, ve, p_end, c),
                    lambda c: c,
                    carry)

                @pl.when(tno + 4 < ntiles)
                def _():
                    pltpu.async_copy(
                        slab_src(tno + 4), slab.at[b], sem_slab)
            return carry

        init = (p_begin - pbase0, pbase0, jnp.int32(0), trash)
        carry = lax.fori_loop(0, (_BASE_T + 4) // 4, loop_body, init)

        def tail_fn(c):
            pltpu.sync_copy(table_hbm.at[:, :, pl.ds(_VCUT, D)], tail_slab)
            return extract_range(tail_slab, _VCUT, VOCAB_SIZE, B, c)

        carry = lax.cond(wid == _NW - 1, tail_fn, lambda c: c, carry)

        _, _, m16, pvec = carry
        pvec = jnp.where(lane < m16, pvec, B)
        flush(pvec)

    run_pass(xin_hbm, sidx_t_hbm, spos_t_hbm, p0_t_hbm, trows_hbm)
    run_pass(xout_hbm, sidx_c_hbm, spos_c_hbm, p0_c_hbm, crows_hbm)


def _dot_body(t_ref, c_ref, o_ref):
    t = t_ref[:, :D]
    c = c_ref[:, :D]
    ones = jnp.ones((D, 1), jnp.float32)
    o_ref[...] = jax.lax.dot(t * c, ones)


_GRID2 = 4
_BLK2 = _OUTROWS // _GRID2
_dot_kernel = pl.pallas_call(
    _dot_body,
    out_shape=jax.ShapeDtypeStruct((_OUTROWS, 1), jnp.float32),
    grid=(_GRID2,),
    in_specs=[
        pl.BlockSpec((_BLK2, 2 * D), lambda i: (i, 0)),
        pl.BlockSpec((_BLK2, 2 * D), lambda i: (i, 0)),
    ],
    out_specs=pl.BlockSpec((_BLK2, 1), lambda i: (i, 0)),
)


def kernel(target_word, context_word, W_in, W_out):
    idx_t = target_word.astype(jnp.int32)
    idx_c = context_word.astype(jnp.int32)
    xin = W_in.T.reshape(8, 8, VOCAB_SIZE)
    xout = W_out.T.reshape(8, 8, VOCAB_SIZE)

    pos = jnp.arange(B, dtype=jnp.int32)
    sidx_t, spos_t = lax.sort((idx_t, pos), num_keys=1)
    sidx_c, spos_c = lax.sort((idx_c, pos), num_keys=1)
    bounds = jnp.asarray(_worker_starts(), dtype=jnp.int32)
    p0_t = jnp.pad(jnp.searchsorted(sidx_t, bounds).astype(jnp.int32),
                   (0, _WIN - _NW - 1))
    p0_c = jnp.pad(jnp.searchsorted(sidx_c, bounds).astype(jnp.int32),
                   (0, _WIN - _NW - 1))
    pad_i = jnp.full((_WIN,), _PAD_IDX, jnp.int32)
    pad_p = jnp.zeros((_WIN,), jnp.int32)
    sidx_t = jnp.concatenate([sidx_t, pad_i])
    spos_t = jnp.concatenate([spos_t, pad_p])
    sidx_c = jnp.concatenate([sidx_c, pad_i])
    spos_c = jnp.concatenate([spos_c, pad_p])

    trows, crows = _stream_kernel(sidx_t, spos_t, p0_t,
                                  sidx_c, spos_c, p0_c, xin, xout)

    return _dot_kernel(trows, crows)[:B, 0]


# speedup vs baseline: 3.5955x; 1.0555x over previous
import functools

import jax
import jax.numpy as jnp
from jax import lax
from jax.experimental import pallas as pl
from jax.experimental.pallas import tpu as pltpu
from jax.experimental.pallas import tpu_sc as plsc

VOCAB_SIZE = 1_000_000
D = 64
B = 16384

_NC = 2
_NS = 16
_L = 16
_NW = _NC * _NS
_VTILE = 128
_NTILES = 7812
_VCUT = _NTILES * _VTILE
_BASE_T = _NTILES // _NW
_EXTRA = _NTILES - _BASE_T * _NW
_WIN = 512
_PAD_IDX = 1 << 20
_OUTROWS = B + 128


def _worker_starts():
    starts = []
    t = 0
    for w in range(_NW + 1):
        starts.append(t * _VTILE)
        t += _BASE_T + (1 if w < _EXTRA else 0)
    starts[_NW] = _VCUT
    return starts


@functools.partial(
    pl.kernel,
    out_type=(
        jax.ShapeDtypeStruct((_OUTROWS, 2 * D), jnp.float32),
        jax.ShapeDtypeStruct((_OUTROWS, 2 * D), jnp.float32),
    ),
    mesh=plsc.VectorSubcoreMesh(core_axis_name="c", subcore_axis_name="s"),
    scratch_types=[
        pltpu.VMEM_SHARED((_NS, _WIN), jnp.int32),
        pltpu.SMEM((_WIN,), jnp.int32),
        pltpu.SMEM((_WIN,), jnp.int32),
        pltpu.VMEM((8, 8, 8, _VTILE), jnp.float32),
        pltpu.VMEM((8, 8, D), jnp.float32),
        pltpu.VMEM((_L, 2 * D), jnp.float32),
        pltpu.SemaphoreType.DMA,
        pltpu.SemaphoreType.DMA,
    ],
    compiler_params=pltpu.CompilerParams(needs_layout_passes=False),
)
def _stream_kernel(sidx_t_hbm, spos_t_hbm, p0_t_hbm,
                   sidx_c_hbm, spos_c_hbm, p0_c_hbm,
                   xin_hbm, xout_hbm, trows_hbm, crows_hbm,
                   sp_hop, win_idx, win_pos, slab, tail_slab, rowblk,
                   sem_slab, sem_out):
    sid = lax.axis_index("s")
    wid = sid * _NC + lax.axis_index("c")
    lane = lax.iota(jnp.int32, _L)
    kvecs = [((jnp.arange(16) + 16 * k) // 8, (jnp.arange(16) + 16 * k) % 8)
             for k in range(4)]
    start_tile = wid * _BASE_T + jnp.minimum(wid, _EXTRA)
    ntiles = _BASE_T + (wid < _EXTRA).astype(jnp.int32)
    trash = jnp.full((_L,), B, jnp.int32)

    def run_pass(table_hbm, sidx_hbm, spos_hbm, p0_hbm, out_hbm):
        pltpu.sync_copy(p0_hbm, sp_hop.at[sid])
        pltpu.sync_copy(sp_hop.at[sid], win_idx)
        p_begin = win_idx[wid]
        p_end = win_idx[wid + 1]

        def refill(pbase):
            pbase = pl.multiple_of(pbase, 128)
            pltpu.sync_copy(sidx_hbm.at[pl.ds(pbase, _WIN)], sp_hop.at[sid])
            pltpu.sync_copy(sp_hop.at[sid], win_idx)
            pltpu.sync_copy(spos_hbm.at[pl.ds(pbase, _WIN)], sp_hop.at[sid])
            pltpu.sync_copy(sp_hop.at[sid], win_pos)

        pbase0 = p_begin & -128
        refill(pbase0)

        def slab_src(tno):
            v0 = (start_tile + tno) * _VTILE
            return table_hbm.at[:, :, pl.ds(v0, _VTILE)]

        def flush(pvec):
            pltpu.async_copy(rowblk, out_hbm.at[pvec], sem_out)
            pltpu.make_async_copy(rowblk, out_hbm.at[pvec], sem_out).wait()

        for b0 in range(8):
            pltpu.async_copy(slab_src(b0), slab.at[b0], sem_slab)

        def extract_range(slab_ref, v0, v_end, p_limit, carry):
            def cond(c):
                pl_o, pb, _, _ = c
                return (pb + pl_o < p_limit) & (win_idx[pl_o] < v_end)

            def body(c):
                pl_o, pb, m, pv = c
                off = jnp.full((_L,), win_idx[pl_o] - v0, jnp.int32)
                for k, (ivec, svec) in enumerate(kvecs):
                    v = plsc.load_gather(slab_ref, [ivec, svec, off])
                    rowblk[m, pl.ds(k * _L, _L)] = v
                pv = jnp.where(lane == m, win_pos[pl_o], pv)
                m = m + 1
                full = m == _L

                @pl.when(full)
                def _():
                    flush(pv)

                m = jnp.where(full, 0, m)
                pl_o = pl_o + 1
                need = pl_o >= _WIN

                @pl.when(need)
                def _():
                    refill(pb + _WIN)

                pl_o = jnp.where(need, pl_o - _WIN, pl_o)
                pb = jnp.where(need, pb + _WIN, pb)
                return pl_o, pb, m, pv

            return lax.while_loop(cond, body, carry)

        def loop_body(g, carry):
            for b in range(8):
                tno = g * 8 + b

                @pl.when(tno < ntiles)
                def _():
                    pltpu.make_async_copy(
                        slab_src(tno), slab.at[b], sem_slab).wait()

                v_end = (start_tile + tno + 1) * _VTILE
                carry = lax.cond(
                    tno < ntiles,
                    lambda c, ve=v_end, bb=b: extract_range(
                        slab.at[bb], ve - _VTILE, ve, p_end, c),
                    lambda c: c,
                    carry)

                @pl.when(tno + 8 < ntiles)
                def _():
                    pltpu.async_copy(
                        slab_src(tno + 8), slab.at[b], sem_slab)
            return carry

        init = (p_begin - pbase0, pbase0, jnp.int32(0), trash)
        carry = lax.fori_loop(0, (_BASE_T + 8) // 8, loop_body, init)

        def tail_fn(c):
            pltpu.sync_copy(table_hbm.at[:, :, pl.ds(_VCUT, D)], tail_slab)
            return extract_range(tail_slab, _VCUT, VOCAB_SIZE, B, c)

        carry = lax.cond(wid == _NW - 1, tail_fn, lambda c: c, carry)

        _, _, m16, pvec = carry
        pvec = jnp.where(lane < m16, pvec, B)
        flush(pvec)

    run_pass(xin_hbm, sidx_t_hbm, spos_t_hbm, p0_t_hbm, trows_hbm)
    run_pass(xout_hbm, sidx_c_hbm, spos_c_hbm, p0_c_hbm, crows_hbm)


def _dot_body(t_ref, c_ref, o_ref):
    t = t_ref[:, :D]
    c = c_ref[:, :D]
    ones = jnp.ones((D, 1), jnp.float32)
    o_ref[...] = jax.lax.dot(t * c, ones)


_GRID2 = 4
_BLK2 = _OUTROWS // _GRID2
_dot_kernel = pl.pallas_call(
    _dot_body,
    out_shape=jax.ShapeDtypeStruct((_OUTROWS, 1), jnp.float32),
    grid=(_GRID2,),
    in_specs=[
        pl.BlockSpec((_BLK2, 2 * D), lambda i: (i, 0)),
        pl.BlockSpec((_BLK2, 2 * D), lambda i: (i, 0)),
    ],
    out_specs=pl.BlockSpec((_BLK2, 1), lambda i: (i, 0)),
)


def kernel(target_word, context_word, W_in, W_out):
    idx_t = target_word.astype(jnp.int32)
    idx_c = context_word.astype(jnp.int32)
    xin = W_in.T.reshape(8, 8, VOCAB_SIZE)
    xout = W_out.T.reshape(8, 8, VOCAB_SIZE)

    pos = jnp.arange(B, dtype=jnp.int32)
    sidx_t, spos_t = lax.sort((idx_t, pos), num_keys=1)
    sidx_c, spos_c = lax.sort((idx_c, pos), num_keys=1)
    bounds = jnp.asarray(_worker_starts(), dtype=jnp.int32)
    p0_t = jnp.pad(jnp.searchsorted(sidx_t, bounds).astype(jnp.int32),
                   (0, _WIN - _NW - 1))
    p0_c = jnp.pad(jnp.searchsorted(sidx_c, bounds).astype(jnp.int32),
                   (0, _WIN - _NW - 1))
    pad_i = jnp.full((_WIN,), _PAD_IDX, jnp.int32)
    pad_p = jnp.zeros((_WIN,), jnp.int32)
    sidx_t = jnp.concatenate([sidx_t, pad_i])
    spos_t = jnp.concatenate([spos_t, pad_p])
    sidx_c = jnp.concatenate([sidx_c, pad_i])
    spos_c = jnp.concatenate([spos_c, pad_p])

    trows, crows = _stream_kernel(sidx_t, spos_t, p0_t,
                                  sidx_c, spos_c, p0_c, xin, xout)

    return _dot_kernel(trows, crows)[:B, 0]
